# Initial kernel scaffold; baseline (speedup 1.0000x reference)
import jax
import jax.numpy as jnp
from jax.experimental import pallas as pl


def kernel(x, edge_index, conv1_W, conv1_u, conv1_c, conv1_b, conv2_W, conv2_u, conv2_c, conv2_b, conv3_W, conv3_u, conv3_c, conv3_b, conv4_W, conv4_u, conv4_c, conv4_b, conv5_W, conv5_u, conv5_c, conv5_b, conv6_W, conv6_u, conv6_c, conv6_b, bn1_g, bn1_b, bn2_g, bn2_b, bn3_g, bn3_b, lin1_W, lin1_b, lin2_W, lin2_b, lin3_W, lin3_b, out_W, out_b):
    raise NotImplementedError("write your pallas kernel here")


# speedup vs baseline: 2.6217x; 2.6217x over previous
import functools

import jax
import jax.numpy as jnp
from jax import lax
from jax.experimental import pallas as pl
from jax.experimental.pallas import tpu as pltpu
from jax.experimental.pallas import tpu_sc as plsc

_L = 16
_NC = 2
_NS = 16
_NW = _NC * _NS
_H = 4
_K = 16
_CW = 128


@functools.lru_cache(maxsize=None)
def _make_sc_edge(c_out, pw, e_pad, npad):
    nv = c_out // _L
    k = _K
    ept = e_pad // _NW
    ch = ept // k
    rpt = npad // _NS
    mesh = plsc.VectorSubcoreMesh(core_axis_name="c", subcore_axis_name="s")

    def body(src_h, dst_h, a_h, crep_h, p_h, s_out,
             s_sh, tab, ctab, sidx, didx, prow, msgb, zb, sem):
        cid = lax.axis_index("c")
        sid = lax.axis_index("s")
        wid = sid * _NC + cid
        zv = jnp.zeros((_L,), jnp.float32)
        onehot = jnp.where(lax.iota(jnp.int32, _L) == 0, 1.0, 0.0)
        onehot = onehot.astype(jnp.float32)
        for r in range(8):
            for v in range(_CW // _L):
                zb[r, pl.ds(v * _L, _L)] = zv
        for r in range(k):
            for v in range(_CW // _L):
                if v == c_out // _L:
                    msgb[r, pl.ds(v * _L, _L)] = onehot
                else:
                    msgb[r, pl.ds(v * _L, _L)] = zv
        pltpu.sync_copy(a_h, tab)
        pltpu.sync_copy(crep_h, ctab)
        row0 = sid * rpt

        def zero_step(i, carry):
            pltpu.sync_copy(zb, s_sh.at[pl.ds(row0 + i * 8, 8)])
            return carry

        lax.fori_loop(0, rpt // 8, zero_step, 0)
        plsc.subcore_barrier()

        base_e = wid * ept

        def chunk(g, carry):
            off = base_e + g * k
            pltpu.sync_copy(src_h.at[pl.ds(off, k)], sidx)
            pltpu.sync_copy(dst_h.at[pl.ds(off, k)], didx)
            pltpu.async_copy(p_h.at[sidx], prow, sem).wait()
            for j0 in range(0, k, _L):
                s16 = sidx[pl.ds(j0, _L)] * _H
                d16 = didx[pl.ds(j0, _L)] * _H
                t = [plsc.load_gather(tab, [s16 + h]) + ctab[h, :] -
                     plsc.load_gather(tab, [d16 + h]) for h in range(_H)]
                m = jnp.maximum(jnp.maximum(t[0], t[1]),
                                jnp.maximum(t[2], t[3]))
                ex = [jnp.exp(tt - m) for tt in t]
                rz = 1.0 / (ex[0] + ex[1] + ex[2] + ex[3])
                q = [exh * rz for exh in ex]
                for j in range(_L):
                    e = j0 + j
                    acc = [prow[e, pl.ds(v * _L, _L)] * q[0][j]
                           for v in range(nv)]
                    for h in range(1, _H):
                        qh = q[h][j]
                        for v in range(nv):
                            acc[v] = acc[v] + (
                                prow[e, pl.ds(h * c_out + v * _L, _L)] * qh)
                    for v in range(nv):
                        msgb[e, pl.ds(v * _L, _L)] = acc[v]
            pltpu.sync_copy(msgb, s_sh.at[didx], add=True)
            return carry

        lax.fori_loop(0, ch, chunk, 0)
        plsc.subcore_barrier()
        pltpu.sync_copy(s_sh.at[pl.ds(row0, rpt)],
                        s_out.at[cid, pl.ds(row0, rpt)])

    return pl.kernel(
        body,
        out_type=(jax.ShapeDtypeStruct((_NC, npad, _CW), jnp.float32),),
        mesh=mesh,
        scratch_types=(
            pltpu.VMEM_SHARED((npad, _CW), jnp.float32),
            pltpu.VMEM((_H * npad,), jnp.float32),
            pltpu.VMEM((_H, _L), jnp.float32),
            pltpu.VMEM((k,), jnp.int32),
            pltpu.VMEM((k,), jnp.int32),
            pltpu.VMEM((k, pw), jnp.float32),
            pltpu.VMEM((k, _CW), jnp.float32),
            pltpu.VMEM((8, _CW), jnp.float32),
            pltpu.SemaphoreType.DMA,
        ),
        compiler_params=pltpu.CompilerParams(needs_layout_passes=False))


def _tc_prep(h, u, W):
    n, cin = h.shape
    hco = W.shape[1]
    bn = 1000

    def body(h_ref, u_ref, w_ref, a_ref, p_ref):
        hb = h_ref[...]
        a_ref[...] = jnp.dot(hb, u_ref[...],
                             preferred_element_type=jnp.float32)
        p_ref[...] = jnp.dot(hb, w_ref[...],
                             preferred_element_type=jnp.float32)

    return pl.pallas_call(
        body,
        grid=(n // bn,),
        in_specs=[pl.BlockSpec((bn, cin), lambda i: (i, 0)),
                  pl.BlockSpec((cin, _H), lambda i: (0, 0)),
                  pl.BlockSpec((cin, hco), lambda i: (0, 0))],
        out_specs=[pl.BlockSpec((bn, _H), lambda i: (i, 0)),
                   pl.BlockSpec((bn, hco), lambda i: (i, 0))],
        out_shape=[jax.ShapeDtypeStruct((n, _H), jnp.float32),
                   jax.ShapeDtypeStruct((n, hco), jnp.float32)],
    )(h, u, W)


def _tc_finish(s_parts, cnt_parts, b, bn_params=None):
    n, c = s_parts.shape[1], s_parts.shape[2]

    if bn_params is None:
        def body(s_ref, cnt_ref, b_ref, o_ref):
            s = s_ref[0] + s_ref[1]
            cnt = cnt_ref[0] + cnt_ref[1]
            o_ref[...] = jnp.maximum(
                s / jnp.maximum(cnt, 1.0) + b_ref[...], 0.0)
        args = (s_parts, cnt_parts, b.reshape(1, -1))
    else:
        g, bb = bn_params

        def body(s_ref, cnt_ref, b_ref, g_ref, bb_ref, o_ref):
            s = s_ref[0] + s_ref[1]
            cnt = cnt_ref[0] + cnt_ref[1]
            hwork = jnp.maximum(s / jnp.maximum(cnt, 1.0) + b_ref[...], 0.0)
            mu = jnp.mean(hwork, axis=0, keepdims=True)
            var = jnp.mean((hwork - mu) ** 2, axis=0, keepdims=True)
            o_ref[...] = ((hwork - mu) / jnp.sqrt(var + 1e-5) * g_ref[...]
                          + bb_ref[...])
        args = (s_parts, cnt_parts, b.reshape(1, -1), g.reshape(1, -1),
                bb.reshape(1, -1))

    return pl.pallas_call(
        body, out_shape=jax.ShapeDtypeStruct((n, c), jnp.float32))(*args)


def _tc_head(x1, x2, x3, w1, b1, w2, b2, w3, b3, wo, bo):
    n = x1.shape[0]
    bs = 2000

    def body(x1r, x2r, x3r, w1r, b1r, w2r, b2r, w3r, b3r, wor, bor, o_ref):
        z = jnp.maximum(
            jnp.concatenate([x1r[...], x2r[...], x3r[...]], axis=1), 0.0)
        z = jnp.maximum(
            jnp.dot(z, w1r[...], preferred_element_type=jnp.float32)
            + b1r[...], 0.0)
        z = jnp.maximum(
            jnp.dot(z, w2r[...], preferred_element_type=jnp.float32)
            + b2r[...], 0.0)
        z = jnp.maximum(
            jnp.dot(z, w3r[...], preferred_element_type=jnp.float32)
            + b3r[...], 0.0)
        z = jnp.dot(z, wor[...], preferred_element_type=jnp.float32) + bor[...]
        o_ref[...] = 1.0 / (1.0 + jnp.exp(-z))

    def full(a):
        return pl.BlockSpec(a.shape, lambda i: (0,) * a.ndim)

    ws = [w1, b1.reshape(1, -1), w2, b2.reshape(1, -1), w3, b3.reshape(1, -1),
          wo, bo.reshape(1, -1)]
    return pl.pallas_call(
        body,
        grid=(n // bs,),
        in_specs=[pl.BlockSpec((bs, x1.shape[1]), lambda i: (i, 0)),
                  pl.BlockSpec((bs, x2.shape[1]), lambda i: (i, 0)),
                  pl.BlockSpec((bs, x3.shape[1]), lambda i: (i, 0))]
                 + [full(a) for a in ws],
        out_specs=pl.BlockSpec((bs, 1), lambda i: (i, 0)),
        out_shape=jax.ShapeDtypeStruct((n, 1), jnp.float32),
    )(x1, x2, x3, *ws)


def kernel(x, edge_index, conv1_W, conv1_u, conv1_c, conv1_b, conv2_W, conv2_u, conv2_c, conv2_b, conv3_W, conv3_u, conv3_c, conv3_b, conv4_W, conv4_u, conv4_c, conv4_b, conv5_W, conv5_u, conv5_c, conv5_b, conv6_W, conv6_u, conv6_c, conv6_b, bn1_g, bn1_b, bn2_g, bn2_b, bn3_g, bn3_b, lin1_W, lin1_b, lin2_W, lin2_b, lin3_W, lin3_b, out_W, out_b):
    n = x.shape[0]
    e = edge_index.shape[1]
    npad = ((n + 1 + _NS * 8 - 1) // (_NS * 8)) * (_NS * 8)
    etot = e + n
    grp = _NW * _K
    e_pad = ((etot + grp - 1) // grp) * grp
    pad = e_pad - etot

    ei = edge_index.astype(jnp.int32)
    loops = jnp.arange(n, dtype=jnp.int32)
    src = jnp.concatenate([ei[0], loops, jnp.zeros((pad,), jnp.int32)])
    dst = jnp.concatenate([ei[1], loops, jnp.full((pad,), n, jnp.int32)])

    cnt_parts = None

    def conv(h, W, u, cvec, first):
        nonlocal cnt_parts
        c_out = W.shape[1] // _H
        pw = ((_H * c_out + 127) // 128) * 128
        w_pad = jnp.pad(W, ((0, 0), (0, pw - _H * c_out)))
        a, p = _tc_prep(h, u, w_pad)
        a_flat = jnp.pad(a, ((0, npad - n), (0, 0))).reshape(-1)
        crep = jnp.tile(cvec.reshape(_H, 1), (1, _L))
        sc = _make_sc_edge(c_out, pw, e_pad, npad)
        (acc,) = sc(src, dst, a_flat, crep, p)
        if first:
            cnt_parts = acc[:, :n, c_out:c_out + 1]
        return acc[:, :n, :c_out]

    s = conv(x, conv1_W, conv1_u, conv1_c, True)
    h1 = _tc_finish(s, cnt_parts, conv1_b)
    s = conv(h1, conv2_W, conv2_u, conv2_c, False)
    x1 = _tc_finish(s, cnt_parts, conv2_b, (bn1_g, bn1_b))
    s = conv(x1, conv3_W, conv3_u, conv3_c, False)
    h3 = _tc_finish(s, cnt_parts, conv3_b)
    s = conv(h3, conv4_W, conv4_u, conv4_c, False)
    x2 = _tc_finish(s, cnt_parts, conv4_b, (bn2_g, bn2_b))
    s = conv(x2, conv5_W, conv5_u, conv5_c, False)
    h5 = _tc_finish(s, cnt_parts, conv5_b)
    s = conv(h5, conv6_W, conv6_u, conv6_c, False)
    x3 = _tc_finish(s, cnt_parts, conv6_b, (bn3_g, bn3_b))
    return _tc_head(x1, x2, x3, lin1_W, lin1_b, lin2_W, lin2_b,
                    lin3_W, lin3_b, out_W, out_b)


# speedup vs baseline: 4.3708x; 1.6672x over previous
import functools

import jax
import jax.numpy as jnp
from jax import lax
from jax.experimental import pallas as pl
from jax.experimental.pallas import tpu as pltpu
from jax.experimental.pallas import tpu_sc as plsc

_L = 16
_NC = 2
_NS = 16
_NW = _NC * _NS
_H = 4
_K = 16
_IB = 256
_CW = 128
_PW = 128


@functools.lru_cache(maxsize=None)
def _make_sc_edge(c_out, with_cnt, e_pad, npad, tabn):
    nv = c_out // _L
    k = _K
    ept = e_pad // _NW
    nblk = ept // _IB
    ncb = _IB // k
    rpt = npad // _NS
    mesh = plsc.VectorSubcoreMesh(core_axis_name="c", subcore_axis_name="s")

    def body(src_h, dst_h, a_h, crep_h, p_h, s_out,
             s_sh, tab, ctab, sidxb, didxb, prow2, msgb2, sem_g, sem_s):
        cid = lax.axis_index("c")
        sid = lax.axis_index("s")
        wid = sid * _NC + cid
        zv = jnp.zeros((_L,), jnp.float32)
        onehot = jnp.where(lax.iota(jnp.int32, _L) == 0, 1.0, 0.0)
        onehot = onehot.astype(jnp.float32)
        for r in range(_L):
            for v in range(_CW // _L):
                msgb2[0, r, pl.ds(v * _L, _L)] = zv
        pltpu.sync_copy(a_h, tab)
        pltpu.sync_copy(crep_h, ctab)
        row0 = sid * rpt

        def zero_step(i, carry):
            pltpu.sync_copy(msgb2.at[0], s_sh.at[pl.ds(row0 + i * _L, _L)])
            return carry

        lax.fori_loop(0, rpt // _L, zero_step, 0)
        rem = rpt % _L
        if rem:
            pltpu.sync_copy(msgb2.at[0].at[pl.ds(0, rem)],
                            s_sh.at[pl.ds(row0 + rpt - rem, rem)])
        plsc.subcore_barrier()
        for b in range(2):
            for r in range(k):
                for v in range(_CW // _L):
                    if v == c_out // _L and with_cnt:
                        msgb2[b, r, pl.ds(v * _L, _L)] = onehot
                    else:
                        msgb2[b, r, pl.ds(v * _L, _L)] = zv

        base_e = wid * ept

        def compute(d16, pb, mb, q):
            for j in range(_L):
                acc = [prow2[pb, j, pl.ds(v * _L, _L)] * q[0][j]
                       for v in range(nv)]
                for h in range(1, _H):
                    qh = q[h][j]
                    for v in range(nv):
                        acc[v] = acc[v] + (
                            prow2[pb, j, pl.ds(h * c_out + v * _L, _L)] * qh)
                for v in range(nv):
                    msgb2[mb, j, pl.ds(v * _L, _L)] = acc[v]

        def softmax(s16, d16):
            t = [plsc.load_gather(tab, [s16 * _H + h]) + ctab[h, :] -
                 plsc.load_gather(tab, [d16 * _H + h]) for h in range(_H)]
            m = jnp.maximum(jnp.maximum(t[0], t[1]), jnp.maximum(t[2], t[3]))
            ex = [jnp.exp(tt - m) for tt in t]
            rz = 1.0 / (ex[0] + ex[1] + ex[2] + ex[3])
            return [exh * rz for exh in ex]

        def block(bi, carry):
            eoff = base_e + bi * _IB
            pltpu.sync_copy(src_h.at[pl.ds(eoff, _IB)], sidxb)
            pltpu.sync_copy(dst_h.at[pl.ds(eoff, _IB)], didxb)

            def pair(pp, c2):
                cc0 = pp * 2
                cc1 = cc0 + 1
                s16_0 = sidxb[pl.ds(cc0 * _L, _L)]
                s16_1 = sidxb[pl.ds(cc1 * _L, _L)]
                d16_0 = didxb[pl.ds(cc0 * _L, _L)]
                d16_1 = didxb[pl.ds(cc1 * _L, _L)]
                g0 = pltpu.async_copy(p_h.at[s16_0], prow2.at[0], sem_g)
                g1 = pltpu.async_copy(p_h.at[s16_1], prow2.at[1], sem_g)
                g0.wait()
                compute(d16_0, 0, 0, softmax(s16_0, d16_0))
                s0 = pltpu.async_copy(msgb2.at[0], s_sh.at[d16_0],
                                      sem_s, add=True)
                g1.wait()
                compute(d16_1, 1, 1, softmax(s16_1, d16_1))
                s1 = pltpu.async_copy(msgb2.at[1], s_sh.at[d16_1],
                                      sem_s, add=True)
                s0.wait()
                s1.wait()
                return c2

            lax.fori_loop(0, ncb // 2, pair, 0)
            return carry

        lax.fori_loop(0, nblk, block, 0)
        plsc.subcore_barrier()
        pltpu.sync_copy(s_sh.at[pl.ds(row0, rpt)],
                        s_out.at[cid, pl.ds(row0, rpt)])

    return pl.kernel(
        body,
        out_type=(jax.ShapeDtypeStruct((_NC, npad, _CW), jnp.float32),),
        mesh=mesh,
        scratch_types=(
            pltpu.VMEM_SHARED((npad, _CW), jnp.float32),
            pltpu.VMEM((tabn,), jnp.float32),
            pltpu.VMEM((_H, _L), jnp.float32),
            pltpu.VMEM((_IB,), jnp.int32),
            pltpu.VMEM((_IB,), jnp.int32),
            pltpu.VMEM((2, _K, _PW), jnp.float32),
            pltpu.VMEM((2, _K, _CW), jnp.float32),
            pltpu.SemaphoreType.DMA,
            pltpu.SemaphoreType.DMA,
        ),
        compiler_params=pltpu.CompilerParams(needs_layout_passes=False))


def _tc_prep(h, u, W):
    n, cin = h.shape
    hco = W.shape[1]
    bn = 1000

    def body(h_ref, u_ref, w_ref, a_ref, p_ref):
        hb = h_ref[...]
        a_ref[...] = jnp.dot(hb, u_ref[...],
                             preferred_element_type=jnp.float32)
        p_ref[...] = jnp.dot(hb, w_ref[...],
                             preferred_element_type=jnp.float32)

    return pl.pallas_call(
        body,
        grid=(n // bn,),
        in_specs=[pl.BlockSpec((bn, cin), lambda i: (i, 0)),
                  pl.BlockSpec((cin, _H), lambda i: (0, 0)),
                  pl.BlockSpec((cin, hco), lambda i: (0, 0))],
        out_specs=[pl.BlockSpec((bn, _H), lambda i: (i, 0)),
                   pl.BlockSpec((bn, hco), lambda i: (i, 0))],
        out_shape=[jax.ShapeDtypeStruct((n, _H), jnp.float32),
                   jax.ShapeDtypeStruct((n, hco), jnp.float32)],
    )(h, u, W)


def _tc_finish(s_parts, cnt_parts, b, bn_params=None):
    n, c = s_parts.shape[1], s_parts.shape[2]

    if bn_params is None:
        def body(s_ref, cnt_ref, b_ref, o_ref):
            s = s_ref[0] + s_ref[1]
            cnt = cnt_ref[0] + cnt_ref[1]
            o_ref[...] = jnp.maximum(
                s / jnp.maximum(cnt, 1.0) + b_ref[...], 0.0)
        args = (s_parts, cnt_parts, b.reshape(1, -1))
    else:
        g, bb = bn_params

        def body(s_ref, cnt_ref, b_ref, g_ref, bb_ref, o_ref):
            s = s_ref[0] + s_ref[1]
            cnt = cnt_ref[0] + cnt_ref[1]
            hwork = jnp.maximum(s / jnp.maximum(cnt, 1.0) + b_ref[...], 0.0)
            mu = jnp.mean(hwork, axis=0, keepdims=True)
            var = jnp.mean((hwork - mu) ** 2, axis=0, keepdims=True)
            o_ref[...] = ((hwork - mu) / jnp.sqrt(var + 1e-5) * g_ref[...]
                          + bb_ref[...])
        args = (s_parts, cnt_parts, b.reshape(1, -1), g.reshape(1, -1),
                bb.reshape(1, -1))

    return pl.pallas_call(
        body, out_shape=jax.ShapeDtypeStruct((n, c), jnp.float32))(*args)


def _tc_head(x1, x2, x3, w1, b1, w2, b2, w3, b3, wo, bo):
    n = x1.shape[0]
    bs = 2000

    def body(x1r, x2r, x3r, w1r, b1r, w2r, b2r, w3r, b3r, wor, bor, o_ref):
        z = jnp.maximum(
            jnp.concatenate([x1r[...], x2r[...], x3r[...]], axis=1), 0.0)
        z = jnp.maximum(
            jnp.dot(z, w1r[...], preferred_element_type=jnp.float32)
            + b1r[...], 0.0)
        z = jnp.maximum(
            jnp.dot(z, w2r[...], preferred_element_type=jnp.float32)
            + b2r[...], 0.0)
        z = jnp.maximum(
            jnp.dot(z, w3r[...], preferred_element_type=jnp.float32)
            + b3r[...], 0.0)
        z = jnp.dot(z, wor[...], preferred_element_type=jnp.float32) + bor[...]
        o_ref[...] = 1.0 / (1.0 + jnp.exp(-z))

    def full(a):
        return pl.BlockSpec(a.shape, lambda i: (0,) * a.ndim)

    ws = [w1, b1.reshape(1, -1), w2, b2.reshape(1, -1), w3, b3.reshape(1, -1),
          wo, bo.reshape(1, -1)]
    return pl.pallas_call(
        body,
        grid=(n // bs,),
        in_specs=[pl.BlockSpec((bs, x1.shape[1]), lambda i: (i, 0)),
                  pl.BlockSpec((bs, x2.shape[1]), lambda i: (i, 0)),
                  pl.BlockSpec((bs, x3.shape[1]), lambda i: (i, 0))]
                 + [full(a) for a in ws],
        out_specs=pl.BlockSpec((bs, 1), lambda i: (i, 0)),
        out_shape=jax.ShapeDtypeStruct((n, 1), jnp.float32),
    )(x1, x2, x3, *ws)


def kernel(x, edge_index, conv1_W, conv1_u, conv1_c, conv1_b, conv2_W, conv2_u, conv2_c, conv2_b, conv3_W, conv3_u, conv3_c, conv3_b, conv4_W, conv4_u, conv4_c, conv4_b, conv5_W, conv5_u, conv5_c, conv5_b, conv6_W, conv6_u, conv6_c, conv6_b, bn1_g, bn1_b, bn2_g, bn2_b, bn3_g, bn3_b, lin1_W, lin1_b, lin2_W, lin2_b, lin3_W, lin3_b, out_W, out_b):
    n = x.shape[0]
    e = edge_index.shape[1]
    npad = ((n + 1 + _NS * 8 - 1) // (_NS * 8)) * (_NS * 8)
    tabn = ((_H * (n + 1) + 7) // 8) * 8
    etot = e + n
    grp = _NW * _IB
    e_pad = ((etot + grp - 1) // grp) * grp
    pad = e_pad - etot

    ei = edge_index.astype(jnp.int32)
    loops = jnp.arange(n, dtype=jnp.int32)
    src = jnp.concatenate([ei[0], loops, jnp.zeros((pad,), jnp.int32)])
    dst = jnp.concatenate([ei[1], loops, jnp.full((pad,), n, jnp.int32)])

    cnt_parts = None

    def run_sc(a_flat, crep, p, c_out, with_cnt):
        sc = _make_sc_edge(c_out, with_cnt, e_pad, npad, tabn)
        (acc,) = sc(src, dst, a_flat, crep, p)
        return acc

    def conv(h, W, u, cvec, first):
        nonlocal cnt_parts
        c_out = W.shape[1] // _H
        crep = jnp.tile(cvec.reshape(_H, 1), (1, _L))
        if c_out <= 32:
            w_pad = jnp.pad(W, ((0, 0), (0, _PW - _H * c_out)))
            a, p = _tc_prep(h, u, w_pad)
            a_flat = jnp.pad(a.reshape(-1), (0, tabn - _H * n))
            acc = run_sc(a_flat, crep, p, c_out, True)
            if first:
                cnt_parts = acc[:, :n, c_out:c_out + 1]
            return acc[:, :n, :c_out]
        half = c_out // 2
        w3d = W.reshape(W.shape[0], _H, c_out)
        w_a = w3d[:, :, :half].reshape(W.shape[0], _H * half)
        w_b = w3d[:, :, half:].reshape(W.shape[0], _H * half)
        a, p_a = _tc_prep(h, u, w_a)
        _, p_b = _tc_prep(h, u, w_b)
        a_flat = jnp.pad(a.reshape(-1), (0, tabn - _H * n))
        acc_a = run_sc(a_flat, crep, p_a, half, True)
        acc_b = run_sc(a_flat, crep, p_b, half, False)
        return jnp.concatenate(
            [acc_a[:, :n, :half], acc_b[:, :n, :half]], axis=2)

    s = conv(x, conv1_W, conv1_u, conv1_c, True)
    h1 = _tc_finish(s, cnt_parts, conv1_b)
    s = conv(h1, conv2_W, conv2_u, conv2_c, False)
    x1 = _tc_finish(s, cnt_parts, conv2_b, (bn1_g, bn1_b))
    s = conv(x1, conv3_W, conv3_u, conv3_c, False)
    h3 = _tc_finish(s, cnt_parts, conv3_b)
    s = conv(h3, conv4_W, conv4_u, conv4_c, False)
    x2 = _tc_finish(s, cnt_parts, conv4_b, (bn2_g, bn2_b))
    s = conv(x2, conv5_W, conv5_u, conv5_c, False)
    h5 = _tc_finish(s, cnt_parts, conv5_b)
    s = conv(h5, conv6_W, conv6_u, conv6_c, False)
    x3 = _tc_finish(s, cnt_parts, conv6_b, (bn3_g, bn3_b))
    return _tc_head(x1, x2, x3, lin1_W, lin1_b, lin2_W, lin2_b,
                    lin3_W, lin3_b, out_W, out_b)


# speedup vs baseline: 4.6684x; 1.0681x over previous
import functools

import jax
import jax.numpy as jnp
from jax import lax
from jax.experimental import pallas as pl
from jax.experimental.pallas import tpu as pltpu
from jax.experimental.pallas import tpu_sc as plsc

_L = 16
_NC = 2
_NS = 16
_NW = _NC * _NS
_H = 4
_K = 16
_IB = 256
_CW = 128
_PW = 128


@functools.lru_cache(maxsize=None)
def _make_sc_edge(c_out, with_cnt, e_pad, npad, tabn):
    nv = c_out // _L
    k = _K
    ept = e_pad // _NW
    nblk = ept // _IB
    ncb = _IB // k
    rpt = npad // _NS
    mesh = plsc.VectorSubcoreMesh(core_axis_name="c", subcore_axis_name="s")

    def body(src_h, dst_h, a_h, crep_h, p_h, s_out,
             s_sh, tab, ctab, sidxb, didxb, prow2, msgb2, sem_g, sem_s):
        cid = lax.axis_index("c")
        sid = lax.axis_index("s")
        wid = sid * _NC + cid
        zv = jnp.zeros((_L,), jnp.float32)
        onehot = jnp.where(lax.iota(jnp.int32, _L) == 0, 1.0, 0.0)
        onehot = onehot.astype(jnp.float32)
        for r in range(_L):
            for v in range(_CW // _L):
                msgb2[0, r, pl.ds(v * _L, _L)] = zv
        pltpu.sync_copy(a_h, tab)
        pltpu.sync_copy(crep_h, ctab)
        row0 = sid * rpt

        def zero_step(i, carry):
            pltpu.sync_copy(msgb2.at[0], s_sh.at[pl.ds(row0 + i * _L, _L)])
            return carry

        lax.fori_loop(0, rpt // _L, zero_step, 0)
        rem = rpt % _L
        if rem:
            pltpu.sync_copy(msgb2.at[0].at[pl.ds(0, rem)],
                            s_sh.at[pl.ds(row0 + rpt - rem, rem)])
        plsc.subcore_barrier()
        for b in range(2):
            for r in range(k):
                for v in range(_CW // _L):
                    if v == c_out // _L and with_cnt:
                        msgb2[b, r, pl.ds(v * _L, _L)] = onehot
                    else:
                        msgb2[b, r, pl.ds(v * _L, _L)] = zv

        base_e = wid * ept

        def compute(d16, pb, mb, q):
            for j in range(_L):
                acc = [prow2[pb, j, pl.ds(v * _L, _L)] * q[0][j]
                       for v in range(nv)]
                for h in range(1, _H):
                    qh = q[h][j]
                    for v in range(nv):
                        acc[v] = acc[v] + (
                            prow2[pb, j, pl.ds(h * c_out + v * _L, _L)] * qh)
                for v in range(nv):
                    msgb2[mb, j, pl.ds(v * _L, _L)] = acc[v]

        def softmax(s16, d16):
            t = [plsc.load_gather(tab, [s16 * _H + h]) + ctab[h, :] -
                 plsc.load_gather(tab, [d16 * _H + h]) for h in range(_H)]
            m = jnp.maximum(jnp.maximum(t[0], t[1]), jnp.maximum(t[2], t[3]))
            ex = [jnp.exp(tt - m) for tt in t]
            rz = 1.0 / (ex[0] + ex[1] + ex[2] + ex[3])
            return [exh * rz for exh in ex]

        def block(bi, carry):
            eoff = base_e + bi * _IB
            pltpu.sync_copy(src_h.at[pl.ds(eoff, _IB)], sidxb)
            pltpu.sync_copy(dst_h.at[pl.ds(eoff, _IB)], didxb)

            def pair(pp, c2):
                cc0 = pp * 2
                cc1 = cc0 + 1
                s16_0 = sidxb[pl.ds(cc0 * _L, _L)]
                s16_1 = sidxb[pl.ds(cc1 * _L, _L)]
                d16_0 = didxb[pl.ds(cc0 * _L, _L)]
                d16_1 = didxb[pl.ds(cc1 * _L, _L)]
                g0 = pltpu.async_copy(p_h.at[s16_0], prow2.at[0], sem_g)
                g1 = pltpu.async_copy(p_h.at[s16_1], prow2.at[1], sem_g)

                @pl.when(jnp.logical_or(bi > 0, pp > 0))
                def _drain():
                    pltpu.make_async_copy(msgb2.at[0], s_sh.at[d16_0],
                                          sem_s).wait()
                    pltpu.make_async_copy(msgb2.at[1], s_sh.at[d16_1],
                                          sem_s).wait()

                g0.wait()
                compute(d16_0, 0, 0, softmax(s16_0, d16_0))
                s0 = pltpu.async_copy(msgb2.at[0], s_sh.at[d16_0],
                                      sem_s, add=True)
                g1.wait()
                compute(d16_1, 1, 1, softmax(s16_1, d16_1))
                s1 = pltpu.async_copy(msgb2.at[1], s_sh.at[d16_1],
                                      sem_s, add=True)
                return c2

            lax.fori_loop(0, ncb // 2, pair, 0)
            return carry

        lax.fori_loop(0, nblk, block, 0)
        dv = sidxb[pl.ds(0, _L)]
        pltpu.make_async_copy(msgb2.at[0], s_sh.at[dv], sem_s).wait()
        pltpu.make_async_copy(msgb2.at[1], s_sh.at[dv], sem_s).wait()
        plsc.subcore_barrier()
        pltpu.sync_copy(s_sh.at[pl.ds(row0, rpt)],
                        s_out.at[cid, pl.ds(row0, rpt)])

    return pl.kernel(
        body,
        out_type=(jax.ShapeDtypeStruct((_NC, npad, _CW), jnp.float32),),
        mesh=mesh,
        scratch_types=(
            pltpu.VMEM_SHARED((npad, _CW), jnp.float32),
            pltpu.VMEM((tabn,), jnp.float32),
            pltpu.VMEM((_H, _L), jnp.float32),
            pltpu.VMEM((_IB,), jnp.int32),
            pltpu.VMEM((_IB,), jnp.int32),
            pltpu.VMEM((2, _K, _PW), jnp.float32),
            pltpu.VMEM((2, _K, _CW), jnp.float32),
            pltpu.SemaphoreType.DMA,
            pltpu.SemaphoreType.DMA,
        ),
        compiler_params=pltpu.CompilerParams(needs_layout_passes=False))


def _tc_prep(h, u, W):
    n, cin = h.shape
    hco = W.shape[1]
    bn = 1000

    def body(h_ref, u_ref, w_ref, a_ref, p_ref):
        hb = h_ref[...]
        a_ref[...] = jnp.dot(hb, u_ref[...],
                             preferred_element_type=jnp.float32)
        p_ref[...] = jnp.dot(hb, w_ref[...],
                             preferred_element_type=jnp.float32)

    return pl.pallas_call(
        body,
        grid=(n // bn,),
        in_specs=[pl.BlockSpec((bn, cin), lambda i: (i, 0)),
                  pl.BlockSpec((cin, _H), lambda i: (0, 0)),
                  pl.BlockSpec((cin, hco), lambda i: (0, 0))],
        out_specs=[pl.BlockSpec((bn, _H), lambda i: (i, 0)),
                   pl.BlockSpec((bn, hco), lambda i: (i, 0))],
        out_shape=[jax.ShapeDtypeStruct((n, _H), jnp.float32),
                   jax.ShapeDtypeStruct((n, hco), jnp.float32)],
    )(h, u, W)


def _tc_finish(s_parts, cnt_parts, b, bn_params=None):
    n, c = s_parts.shape[1], s_parts.shape[2]

    if bn_params is None:
        def body(s_ref, cnt_ref, b_ref, o_ref):
            s = s_ref[0] + s_ref[1]
            cnt = cnt_ref[0] + cnt_ref[1]
            o_ref[...] = jnp.maximum(
                s / jnp.maximum(cnt, 1.0) + b_ref[...], 0.0)
        args = (s_parts, cnt_parts, b.reshape(1, -1))
    else:
        g, bb = bn_params

        def body(s_ref, cnt_ref, b_ref, g_ref, bb_ref, o_ref):
            s = s_ref[0] + s_ref[1]
            cnt = cnt_ref[0] + cnt_ref[1]
            hwork = jnp.maximum(s / jnp.maximum(cnt, 1.0) + b_ref[...], 0.0)
            mu = jnp.mean(hwork, axis=0, keepdims=True)
            var = jnp.mean((hwork - mu) ** 2, axis=0, keepdims=True)
            o_ref[...] = ((hwork - mu) / jnp.sqrt(var + 1e-5) * g_ref[...]
                          + bb_ref[...])
        args = (s_parts, cnt_parts, b.reshape(1, -1), g.reshape(1, -1),
                bb.reshape(1, -1))

    return pl.pallas_call(
        body, out_shape=jax.ShapeDtypeStruct((n, c), jnp.float32))(*args)


def _tc_head(x1, x2, x3, w1, b1, w2, b2, w3, b3, wo, bo):
    n = x1.shape[0]
    bs = 2000

    def body(x1r, x2r, x3r, w1r, b1r, w2r, b2r, w3r, b3r, wor, bor, o_ref):
        z = jnp.maximum(
            jnp.concatenate([x1r[...], x2r[...], x3r[...]], axis=1), 0.0)
        z = jnp.maximum(
            jnp.dot(z, w1r[...], preferred_element_type=jnp.float32)
            + b1r[...], 0.0)
        z = jnp.maximum(
            jnp.dot(z, w2r[...], preferred_element_type=jnp.float32)
            + b2r[...], 0.0)
        z = jnp.maximum(
            jnp.dot(z, w3r[...], preferred_element_type=jnp.float32)
            + b3r[...], 0.0)
        z = jnp.dot(z, wor[...], preferred_element_type=jnp.float32) + bor[...]
        o_ref[...] = 1.0 / (1.0 + jnp.exp(-z))

    def full(a):
        return pl.BlockSpec(a.shape, lambda i: (0,) * a.ndim)

    ws = [w1, b1.reshape(1, -1), w2, b2.reshape(1, -1), w3, b3.reshape(1, -1),
          wo, bo.reshape(1, -1)]
    return pl.pallas_call(
        body,
        grid=(n // bs,),
        in_specs=[pl.BlockSpec((bs, x1.shape[1]), lambda i: (i, 0)),
                  pl.BlockSpec((bs, x2.shape[1]), lambda i: (i, 0)),
                  pl.BlockSpec((bs, x3.shape[1]), lambda i: (i, 0))]
                 + [full(a) for a in ws],
        out_specs=pl.BlockSpec((bs, 1), lambda i: (i, 0)),
        out_shape=jax.ShapeDtypeStruct((n, 1), jnp.float32),
    )(x1, x2, x3, *ws)


def kernel(x, edge_index, conv1_W, conv1_u, conv1_c, conv1_b, conv2_W, conv2_u, conv2_c, conv2_b, conv3_W, conv3_u, conv3_c, conv3_b, conv4_W, conv4_u, conv4_c, conv4_b, conv5_W, conv5_u, conv5_c, conv5_b, conv6_W, conv6_u, conv6_c, conv6_b, bn1_g, bn1_b, bn2_g, bn2_b, bn3_g, bn3_b, lin1_W, lin1_b, lin2_W, lin2_b, lin3_W, lin3_b, out_W, out_b):
    n = x.shape[0]
    e = edge_index.shape[1]
    npad = ((n + 1 + _NS * 8 - 1) // (_NS * 8)) * (_NS * 8)
    tabn = ((_H * (n + 1) + 7) // 8) * 8
    etot = e + n
    grp = _NW * _IB
    e_pad = ((etot + grp - 1) // grp) * grp
    pad = e_pad - etot

    ei = edge_index.astype(jnp.int32)
    loops = jnp.arange(n, dtype=jnp.int32)
    src = jnp.concatenate([ei[0], loops, jnp.zeros((pad,), jnp.int32)])
    dst = jnp.concatenate([ei[1], loops, jnp.full((pad,), n, jnp.int32)])

    cnt_parts = None

    def run_sc(a_flat, crep, p, c_out, with_cnt):
        sc = _make_sc_edge(c_out, with_cnt, e_pad, npad, tabn)
        (acc,) = sc(src, dst, a_flat, crep, p)
        return acc

    def conv(h, W, u, cvec, first):
        nonlocal cnt_parts
        c_out = W.shape[1] // _H
        crep = jnp.tile(cvec.reshape(_H, 1), (1, _L))
        if c_out <= 32:
            w_pad = jnp.pad(W, ((0, 0), (0, _PW - _H * c_out)))
            a, p = _tc_prep(h, u, w_pad)
            a_flat = jnp.pad(a.reshape(-1), (0, tabn - _H * n))
            acc = run_sc(a_flat, crep, p, c_out, True)
            if first:
                cnt_parts = acc[:, :n, c_out:c_out + 1]
            return acc[:, :n, :c_out]
        half = c_out // 2
        w3d = W.reshape(W.shape[0], _H, c_out)
        w_a = w3d[:, :, :half].reshape(W.shape[0], _H * half)
        w_b = w3d[:, :, half:].reshape(W.shape[0], _H * half)
        a, p_a = _tc_prep(h, u, w_a)
        _, p_b = _tc_prep(h, u, w_b)
        a_flat = jnp.pad(a.reshape(-1), (0, tabn - _H * n))
        acc_a = run_sc(a_flat, crep, p_a, half, True)
        acc_b = run_sc(a_flat, crep, p_b, half, False)
        return jnp.concatenate(
            [acc_a[:, :n, :half], acc_b[:, :n, :half]], axis=2)

    s = conv(x, conv1_W, conv1_u, conv1_c, True)
    h1 = _tc_finish(s, cnt_parts, conv1_b)
    s = conv(h1, conv2_W, conv2_u, conv2_c, False)
    x1 = _tc_finish(s, cnt_parts, conv2_b, (bn1_g, bn1_b))
    s = conv(x1, conv3_W, conv3_u, conv3_c, False)
    h3 = _tc_finish(s, cnt_parts, conv3_b)
    s = conv(h3, conv4_W, conv4_u, conv4_c, False)
    x2 = _tc_finish(s, cnt_parts, conv4_b, (bn2_g, bn2_b))
    s = conv(x2, conv5_W, conv5_u, conv5_c, False)
    h5 = _tc_finish(s, cnt_parts, conv5_b)
    s = conv(h5, conv6_W, conv6_u, conv6_c, False)
    x3 = _tc_finish(s, cnt_parts, conv6_b, (bn3_g, bn3_b))
    return _tc_head(x1, x2, x3, lin1_W, lin1_b, lin2_W, lin2_b,
                    lin3_W, lin3_b, out_W, out_b)
